# Initial kernel scaffold; baseline (speedup 1.0000x reference)
import jax
import jax.numpy as jnp
from jax.experimental import pallas as pl


def kernel(x, centroids):
    raise NotImplementedError("write your pallas kernel here")


# speedup vs baseline: 1.3526x; 1.3526x over previous
import functools

import jax
import jax.numpy as jnp
from jax import lax
from jax.experimental import pallas as pl
from jax.experimental.pallas import tpu as pltpu
from jax.experimental.pallas import tpu_sc as plsc

_NUM_FEATURES = 256
_NUM_CLUSTERS = 8192
_EPS = 1e-6

_BT = 1024
_BK = 2048

_SC_CORES = 2
_SC_SUBCORES = 16
_SC_WORKERS = _SC_CORES * _SC_SUBCORES


def _assign_body(x_ref, c_ref, lab_ref, min_ref, arg_ref):
    j = pl.program_id(1)
    nj = pl.num_programs(1)

    @pl.when(j == 0)
    def _():
        min_ref[...] = jnp.full_like(min_ref, jnp.inf)
        arg_ref[...] = jnp.zeros_like(arg_ref)

    xb = x_ref[...]
    cb = c_ref[...]
    d = xb.shape[1]
    x_sq = jnp.sum(xb * xb, axis=1, keepdims=True)
    x_sum = jnp.sum(xb, axis=1, keepdims=True)
    c_sq = jnp.sum(cb * cb, axis=1)[None, :]
    c_sum = jnp.sum(cb, axis=1)[None, :]
    cross = lax.dot_general(xb, cb, (((1,), (1,)), ((), ())),
                            preferred_element_type=jnp.float32)
    sq = x_sq + c_sq - 2.0 * cross + 2.0 * _EPS * (x_sum - c_sum) + d * _EPS * _EPS

    m = jnp.min(sq, axis=1, keepdims=True)
    col = lax.broadcasted_iota(jnp.int32, sq.shape, 1)
    idx = jnp.min(jnp.where(sq == m, col, sq.shape[1]), axis=1, keepdims=True)
    idx = idx + j * sq.shape[1]

    better = m < min_ref[...]
    arg_ref[...] = jnp.where(better, idx, arg_ref[...])
    min_ref[...] = jnp.where(better, m, min_ref[...])

    @pl.when(j == nj - 1)
    def _():
        lab_ref[...] = arg_ref[...]


def _assign_labels(xf, centroids):
    n = xf.shape[0]
    k = centroids.shape[0]
    grid = (n // _BT, k // _BK)
    return pl.pallas_call(
        _assign_body,
        grid=grid,
        in_specs=[
            pl.BlockSpec((_BT, _NUM_FEATURES), lambda t, j: (t, 0)),
            pl.BlockSpec((_BK, _NUM_FEATURES), lambda t, j: (j, 0)),
        ],
        out_specs=pl.BlockSpec((_BT, 1), lambda t, j: (t, 0)),
        out_shape=jax.ShapeDtypeStruct((n, 1), jnp.int32),
        scratch_shapes=[
            pltpu.VMEM((_BT, 1), jnp.float32),
            pltpu.VMEM((_BT, 1), jnp.int32),
        ],
        compiler_params=pltpu.CompilerParams(
            dimension_semantics=("parallel", "arbitrary"),
        ),
    )(xf, centroids)


def _gather_body(table_hbm, idx_hbm, out_hbm, idx_v, rows_v, sem):
    wid = lax.axis_index("s") * _SC_CORES + lax.axis_index("c")
    bpw = idx_v.shape[0]
    base = wid * bpw
    pltpu.sync_copy(idx_hbm.at[pl.ds(base, bpw)], idx_v)
    pltpu.async_copy(table_hbm.at[idx_v], rows_v, sem).wait()
    pltpu.sync_copy(rows_v, out_hbm.at[pl.ds(base, bpw)])


def _gather_rows(centroids, labels):
    n = labels.shape[0]
    bpw = n // _SC_WORKERS
    mesh = plsc.VectorSubcoreMesh(core_axis_name="c", subcore_axis_name="s")
    return pl.kernel(
        _gather_body,
        out_type=jax.ShapeDtypeStruct((n, _NUM_FEATURES), jnp.float32),
        mesh=mesh,
        scratch_types=[
            pltpu.VMEM((bpw,), jnp.int32),
            pltpu.VMEM((bpw, _NUM_FEATURES), jnp.float32),
            pltpu.SemaphoreType.DMA,
        ],
    )(centroids, labels)


def kernel(x, centroids):
    batch_shape = x.shape[:-1]
    nf = centroids.shape[-1]
    xf = x.reshape(-1, nf)
    labels2d = _assign_labels(xf, centroids)
    labels = labels2d.reshape(-1)
    assigned = _gather_rows(centroids, labels)
    return labels.reshape(batch_shape), assigned.reshape(batch_shape + (nf,))


# speedup vs baseline: 2.2074x; 1.6320x over previous
import functools

import jax
import jax.numpy as jnp
from jax import lax
from jax.experimental import pallas as pl
from jax.experimental.pallas import tpu as pltpu
from jax.experimental.pallas import tpu_sc as plsc

_NUM_FEATURES = 256
_NUM_CLUSTERS = 8192
_EPS = 1e-6

_BT = 1024
_BK = 2048

_SC_CORES = 2
_SC_SUBCORES = 16
_SC_WORKERS = _SC_CORES * _SC_SUBCORES


def _assign_body(x_ref, c_ref, lab_ref, val_ref, chunk_ref, bias_ref):
    t = pl.program_id(0)
    j = pl.program_id(1)
    nj = pl.num_programs(1)

    @pl.when(j == 0)
    def _():
        val_ref[...] = jnp.full_like(val_ref, jnp.inf)
        chunk_ref[...] = jnp.zeros_like(chunk_ref)

    xb = x_ref[...]
    cb = c_ref[...]
    bk = cb.shape[0]

    @pl.when(t == 0)
    def _():
        c_sq = jnp.sum(cb * cb, axis=1)[None, :]
        c_sum = jnp.sum(cb, axis=1)[None, :]
        bias_ref[:, pl.ds(j * bk, bk)] = c_sq - (2.0 * _EPS) * c_sum

    cross = lax.dot_general(xb, cb, (((1,), (1,)), ((), ())),
                            preferred_element_type=jnp.float32)
    score = bias_ref[:, pl.ds(j * bk, bk)] - 2.0 * cross

    run_val = val_ref[...]
    run_chunk = chunk_ref[...]
    for c in range(bk // 128):
        chunk = score[:, c * 128:(c + 1) * 128]
        better = chunk < run_val
        run_val = jnp.where(better, chunk, run_val)
        run_chunk = jnp.where(better, j * (bk // 128) + c, run_chunk)
    val_ref[...] = run_val
    chunk_ref[...] = run_chunk

    @pl.when(j == nj - 1)
    def _():
        lane = lax.broadcasted_iota(jnp.int32, run_chunk.shape, 1)
        gidx = run_chunk * 128 + lane
        rowmin = jnp.min(run_val, axis=1, keepdims=True)
        big = jnp.int32(_NUM_CLUSTERS)
        lab_ref[...] = jnp.min(jnp.where(run_val == rowmin, gidx, big),
                               axis=1, keepdims=True)


def _assign_labels(xf, centroids):
    n = xf.shape[0]
    k = centroids.shape[0]
    grid = (n // _BT, k // _BK)
    return pl.pallas_call(
        _assign_body,
        grid=grid,
        in_specs=[
            pl.BlockSpec((_BT, _NUM_FEATURES), lambda t, j: (t, 0)),
            pl.BlockSpec((_BK, _NUM_FEATURES), lambda t, j: (j, 0)),
        ],
        out_specs=pl.BlockSpec((_BT, 1), lambda t, j: (t, 0)),
        out_shape=jax.ShapeDtypeStruct((n, 1), jnp.int32),
        scratch_shapes=[
            pltpu.VMEM((_BT, 128), jnp.float32),
            pltpu.VMEM((_BT, 128), jnp.int32),
            pltpu.VMEM((1, k), jnp.float32),
        ],
        compiler_params=pltpu.CompilerParams(
            dimension_semantics=("arbitrary", "arbitrary"),
        ),
    )(xf, centroids)


def _gather_body(table_hbm, idx_hbm, out_hbm, idx_v, rows_v, sem):
    wid = lax.axis_index("s") * _SC_CORES + lax.axis_index("c")
    bpw = idx_v.shape[0]
    base = wid * bpw
    pltpu.sync_copy(idx_hbm.at[pl.ds(base, bpw)], idx_v)
    pltpu.async_copy(table_hbm.at[idx_v], rows_v, sem).wait()
    pltpu.sync_copy(rows_v, out_hbm.at[pl.ds(base, bpw)])


def _gather_rows(centroids, labels):
    n = labels.shape[0]
    bpw = n // _SC_WORKERS
    mesh = plsc.VectorSubcoreMesh(core_axis_name="c", subcore_axis_name="s")
    return pl.kernel(
        _gather_body,
        out_type=jax.ShapeDtypeStruct((n, _NUM_FEATURES), jnp.float32),
        mesh=mesh,
        scratch_types=[
            pltpu.VMEM((bpw,), jnp.int32),
            pltpu.VMEM((bpw, _NUM_FEATURES), jnp.float32),
            pltpu.SemaphoreType.DMA,
        ],
    )(centroids, labels)


def kernel(x, centroids):
    batch_shape = x.shape[:-1]
    nf = centroids.shape[-1]
    xf = x.reshape(-1, nf)
    labels2d = _assign_labels(xf, centroids)
    labels = labels2d.reshape(-1)
    assigned = _gather_rows(centroids, labels)
    return labels.reshape(batch_shape), assigned.reshape(batch_shape + (nf,))


# speedup vs baseline: 2.3947x; 1.0848x over previous
import functools

import jax
import jax.numpy as jnp
from jax import lax
from jax.experimental import pallas as pl
from jax.experimental.pallas import tpu as pltpu
from jax.experimental.pallas import tpu_sc as plsc

_NUM_FEATURES = 256
_NUM_CLUSTERS = 8192
_EPS = 1e-6

_BT = 1024
_BK = 2048

_SC_CORES = 2
_SC_SUBCORES = 16
_SC_WORKERS = _SC_CORES * _SC_SUBCORES


def _assign_body(x_ref, c_ref, lab_ref, val_ref, chunk_ref, bias_ref):
    t = pl.program_id(0)
    j = pl.program_id(1)
    nj = pl.num_programs(1)

    @pl.when(j == 0)
    def _():
        val_ref[...] = jnp.full_like(val_ref, jnp.inf)
        chunk_ref[...] = jnp.zeros_like(chunk_ref)

    xb = x_ref[...]
    cb = c_ref[...]
    bk = cb.shape[0]

    @pl.when(t == 0)
    def _():
        c_sq = jnp.sum(cb * cb, axis=1)[None, :]
        c_sum = jnp.sum(cb, axis=1)[None, :]
        bias_ref[:, pl.ds(j * bk, bk)] = c_sq - (2.0 * _EPS) * c_sum

    cross2 = lax.dot_general(xb, cb * (-2.0), (((1,), (1,)), ((), ())),
                             preferred_element_type=jnp.float32)

    run_val = val_ref[...]
    run_chunk = chunk_ref[...]
    for c in range(bk // 128):
        chunk = bias_ref[:, pl.ds(j * bk + c * 128, 128)] + \
            cross2[:, c * 128:(c + 1) * 128]
        better = chunk < run_val
        run_val = jnp.where(better, chunk, run_val)
        run_chunk = jnp.where(better, j * (bk // 128) + c, run_chunk)
    val_ref[...] = run_val
    chunk_ref[...] = run_chunk

    @pl.when(j == nj - 1)
    def _():
        lane = lax.broadcasted_iota(jnp.int32, run_chunk.shape, 1)
        gidx = run_chunk * 128 + lane
        rowmin = jnp.min(run_val, axis=1, keepdims=True)
        big = jnp.int32(_NUM_CLUSTERS)
        lab_ref[...] = jnp.min(jnp.where(run_val == rowmin, gidx, big),
                               axis=1, keepdims=True)


def _assign_labels(xf, centroids):
    n = xf.shape[0]
    k = centroids.shape[0]
    grid = (n // _BT, k // _BK)
    return pl.pallas_call(
        _assign_body,
        grid=grid,
        in_specs=[
            pl.BlockSpec((_BT, _NUM_FEATURES), lambda t, j: (t, 0)),
            pl.BlockSpec((_BK, _NUM_FEATURES), lambda t, j: (j, 0)),
        ],
        out_specs=pl.BlockSpec((_BT, 1), lambda t, j: (t, 0)),
        out_shape=jax.ShapeDtypeStruct((n, 1), jnp.int32),
        scratch_shapes=[
            pltpu.VMEM((_BT, 128), jnp.float32),
            pltpu.VMEM((_BT, 128), jnp.int32),
            pltpu.VMEM((1, k), jnp.float32),
        ],
        compiler_params=pltpu.CompilerParams(
            dimension_semantics=("arbitrary", "arbitrary"),
        ),
    )(xf, centroids)


def _gather_body(table_hbm, idx_hbm, out_hbm, idx_v, rows_v, sem):
    wid = lax.axis_index("s") * _SC_CORES + lax.axis_index("c")
    bpw = idx_v.shape[0]
    base = wid * bpw
    pltpu.sync_copy(idx_hbm.at[pl.ds(base, bpw)], idx_v)
    pltpu.async_copy(table_hbm.at[idx_v], rows_v, sem).wait()
    pltpu.sync_copy(rows_v, out_hbm.at[pl.ds(base, bpw)])


def _gather_rows(centroids, labels):
    n = labels.shape[0]
    bpw = n // _SC_WORKERS
    mesh = plsc.VectorSubcoreMesh(core_axis_name="c", subcore_axis_name="s")
    return pl.kernel(
        _gather_body,
        out_type=jax.ShapeDtypeStruct((n, _NUM_FEATURES), jnp.float32),
        mesh=mesh,
        scratch_types=[
            pltpu.VMEM((bpw,), jnp.int32),
            pltpu.VMEM((bpw, _NUM_FEATURES), jnp.float32),
            pltpu.SemaphoreType.DMA,
        ],
    )(centroids, labels)


def kernel(x, centroids):
    batch_shape = x.shape[:-1]
    nf = centroids.shape[-1]
    xf = x.reshape(-1, nf)
    labels2d = _assign_labels(xf, centroids)
    labels = labels2d.reshape(-1)
    assigned = _gather_rows(centroids, labels)
    return labels.reshape(batch_shape), assigned.reshape(batch_shape + (nf,))
